# Initial kernel scaffold; baseline (speedup 1.0000x reference)
import jax
import jax.numpy as jnp
from jax.experimental import pallas as pl


def kernel(tokens, We1, be1, Wa1, ba1, Wa2, ba2, Wk1, bk1, Wk2, bk2, Wp1, bp1, Wp2, bp2, Wr1, br1, Wr2, br2, Wf1, bf1, Wf2, bf2):
    raise NotImplementedError("write your pallas kernel here")


# speedup vs baseline: 2.3808x; 2.3808x over previous
import functools

import jax
import jax.numpy as jnp
from jax import lax
from jax.experimental import pallas as pl
from jax.experimental.pallas import tpu as pltpu

_MAXK = 20


def _score_sum_body(tok_ref, wp1_ref, bp1_ref, wp2_ref, imp_ref, sum_ref):
    i = pl.program_id(0)
    x = tok_ref[...]
    B, BN, C = x.shape
    x2 = x.reshape(B * BN, C)
    h = jnp.maximum(
        jnp.dot(x2, wp1_ref[...], preferred_element_type=jnp.float32)
        + bp1_ref[...], 0.0)
    imp = jnp.sum(h * wp2_ref[...], axis=1)
    imp_ref[...] = imp.reshape(B, BN)

    @pl.when(i == 0)
    def _():
        sum_ref[...] = jnp.zeros_like(sum_ref)

    sum_ref[...] += jnp.sum(x, axis=1)


def _score_and_sum(tokens, Wp1, bp1, Wp2, block_n=1024):
    B, N, C = tokens.shape
    hid = Wp1.shape[1]
    grid = (N // block_n,)
    imp, sums = pl.pallas_call(
        _score_sum_body,
        grid=grid,
        in_specs=[
            pl.BlockSpec((B, block_n, C), lambda i: (0, i, 0)),
            pl.BlockSpec((C, hid), lambda i: (0, 0)),
            pl.BlockSpec((1, hid), lambda i: (0, 0)),
            pl.BlockSpec((1, hid), lambda i: (0, 0)),
        ],
        out_specs=[
            pl.BlockSpec((B, block_n), lambda i: (0, i)),
            pl.BlockSpec((B, C), lambda i: (0, 0)),
        ],
        out_shape=[
            jax.ShapeDtypeStruct((B, N), jnp.float32),
            jax.ShapeDtypeStruct((B, C), jnp.float32),
        ],
    )(tokens, Wp1, bp1.reshape(1, hid), Wp2.reshape(1, hid))
    return imp, sums


def _finalize_body(sums_ref, tt_ref,
                   we1_ref, be1_ref, wa1_ref, ba1_ref, wa2_ref, ba2_ref,
                   wk1_ref, bk1_ref, wk2_ref, bk2_ref,
                   wr1_ref, br1_ref, wr2_ref, br2_ref,
                   wf1_ref, bf1_ref, wf2_ref, bf2_ref,
                   n_ref, out_ref):
    B = sums_ref.shape[0]
    nf = n_ref[0, 0]
    sums = sums_ref[...]
    mean = sums / nf
    feat = jnp.maximum(
        jnp.dot(mean, we1_ref[...], preferred_element_type=jnp.float32)
        + be1_ref[...], 0.0)
    a1 = jnp.maximum(
        jnp.dot(feat, wa1_ref[...], preferred_element_type=jnp.float32)
        + ba1_ref[...], 0.0)
    alpha = jax.nn.sigmoid(
        jnp.dot(a1, wa2_ref[...], preferred_element_type=jnp.float32)
        + ba2_ref[...])
    k1 = jnp.maximum(
        jnp.dot(feat, wk1_ref[...], preferred_element_type=jnp.float32)
        + bk1_ref[...], 0.0)
    kz = jnp.dot(k1, wk2_ref[...], preferred_element_type=jnp.float32) + bk2_ref[...]
    kraw = jnp.maximum(kz, 0.0) + jnp.log1p(jnp.exp(-jnp.abs(kz)))
    kf = jnp.clip(jnp.round(kraw), 1.0, float(_MAXK))
    takf = jnp.maximum(1.0, jnp.floor(alpha * kf))

    tt = tt_ref[...]
    hh = jnp.maximum(
        jnp.dot(tt, wr1_ref[...], preferred_element_type=jnp.float32)
        + br1_ref[...], 0.0)
    refined = jnp.dot(hh, wr2_ref[...], preferred_element_type=jnp.float32) \
        + br2_ref[...]

    pos = lax.broadcasted_iota(jnp.int32, (_MAXK, 1), 0).astype(jnp.float32)
    aggs = []
    for b in range(B):
        tak_b = lax.slice(takf, (b, 0), (b + 1, 1))
        m_b = pos < tak_b
        ref_b = refined[b * _MAXK:(b + 1) * _MAXK, :]
        tt_b = tt[b * _MAXK:(b + 1) * _MAXK, :]
        refined_sum = jnp.sum(jnp.where(m_b, ref_b, 0.0), axis=0, keepdims=True)
        top_sum = jnp.sum(jnp.where(m_b, tt_b, 0.0), axis=0, keepdims=True)
        pooled = (sums[b:b + 1, :] - top_sum) / (nf - tak_b)
        aggs.append((refined_sum + pooled) / (tak_b + 1.0))
    agg = jnp.concatenate(aggs, axis=0)

    fh = jnp.maximum(
        jnp.dot(agg, wf1_ref[...], preferred_element_type=jnp.float32)
        + bf1_ref[...], 0.0)
    out_ref[...] = jnp.dot(fh, wf2_ref[...], preferred_element_type=jnp.float32) \
        + bf2_ref[...]


def _finalize(sums, tt, n,
              We1, be1, Wa1, ba1, Wa2, ba2, Wk1, bk1, Wk2, bk2,
              Wr1, br1, Wr2, br2, Wf1, bf1, Wf2, bf2):
    B, C = sums.shape
    hid = We1.shape[1]
    h2 = Wa1.shape[1]
    args = (
        sums, tt,
        We1, be1.reshape(1, hid), Wa1, ba1.reshape(1, h2),
        Wa2, ba2.reshape(1, 1), Wk1, bk1.reshape(1, h2), Wk2, bk2.reshape(1, 1),
        Wr1, br1.reshape(1, hid), Wr2, br2.reshape(1, C),
        Wf1, bf1.reshape(1, hid), Wf2, bf2.reshape(1, C),
        jnp.full((1, 1), float(n), jnp.float32),
    )
    return pl.pallas_call(
        _finalize_body,
        out_shape=jax.ShapeDtypeStruct((B, C), jnp.float32),
    )(*args)


def kernel(tokens, We1, be1, Wa1, ba1, Wa2, ba2, Wk1, bk1, Wk2, bk2,
           Wp1, bp1, Wp2, bp2, Wr1, br1, Wr2, br2, Wf1, bf1, Wf2, bf2):
    B, N, C = tokens.shape
    imp, sums = _score_and_sum(tokens, Wp1, bp1, Wp2)
    _, top_idx = lax.top_k(imp, _MAXK)
    tt = jnp.take_along_axis(tokens, top_idx[..., None], axis=1)
    tt = tt.reshape(B * _MAXK, C)
    return _finalize(sums, tt, N,
                     We1, be1, Wa1, ba1, Wa2, ba2, Wk1, bk1, Wk2, bk2,
                     Wr1, br1, Wr2, br2, Wf1, bf1, Wf2, bf2)
